# baseline (device time: 26067 ns/iter reference)
import jax
import jax.numpy as jnp
from jax import lax
from jax.experimental import pallas as pl
from jax.experimental.pallas import tpu as pltpu

_BITS = list(range(10, 2, -1))


def _pos_kernel(dest2d, m):

    def body(d_ref, pos_ref, a_ref):
        mz = lax.axis_index("z")
        d = d_ref[...]
        rows, lanes = d.shape

        p = d
        lane = lax.broadcasted_iota(jnp.int32, (rows, lanes), 1)
        for sh in (1, 2, 4, 8, 16, 32, 64):
            p = p + jnp.where(lane >= sh, pltpu.roll(p, sh, 1), 0)
        rowtot = jnp.broadcast_to(p[:, lanes - 1:lanes], (rows, lanes))
        row = lax.broadcasted_iota(jnp.int32, (rows, lanes), 0)
        b = rowtot
        for sh in (1, 2, 4):
            b = b + jnp.where(row >= sh, pltpu.roll(b, sh, 0), 0)
        c = p + (b - rowtot)
        n1 = jnp.max(c)
        n0 = m - n1
        a = jnp.where(mz == 0, n1, n0)
        e = (-a) % 8

        i = row * lanes + lane
        z = i + 1 - c
        keep_pos = jnp.where(mz == 0, z - 1, n0 + c - 1)
        send_rank = jnp.where(mz == 0, c - 1, z - 1)
        send_off = jnp.where(mz == 0, 0, e)
        pos_ref[...] = jnp.where(d == mz, keep_pos, m + send_off + send_rank)
        a_ref[0] = a

    return pl.pallas_call(
        body,
        out_shape=(
            jax.ShapeDtypeStruct(dest2d.shape, jnp.int32),
            jax.ShapeDtypeStruct((1,), jnp.int32),
        ),
        in_specs=[pl.BlockSpec(memory_space=pltpu.VMEM)],
        out_specs=(
            pl.BlockSpec(memory_space=pltpu.VMEM),
            pl.BlockSpec(memory_space=pltpu.SMEM),
        ),
    )(dest2d)


def kernel(x, dest):
    m, n = x.shape

    pos2d, a_arr = _pos_kernel(dest.reshape(8, -1), m)
    pos = pos2d.reshape(m)
    sall = jnp.zeros((2 * m, n), x.dtype).at[pos].set(
        x, mode="drop", unique_indices=True)

    def body(a_ref, sall_ref, out_ref, send_sems, recv_sems, copy_sems):
        mx = lax.axis_index("x")
        my = lax.axis_index("y")
        mz = lax.axis_index("z")
        peer = (mx, my, 1 - mz)
        a = a_ref[0]
        e = (-a) % 8
        A = a + e
        rbase = jnp.where(mz == 0, m - A, 0)
        dst0 = jnp.where(mz == 0, 0, m - A)
        lbase = jnp.where(mz == 0, 0, A)

        barrier_sem = pltpu.get_barrier_semaphore()
        pl.semaphore_signal(
            barrier_sem, inc=1, device_id=peer,
            device_id_type=pl.DeviceIdType.MESH,
        )

        copies = []
        loff = jnp.int32(0)
        klen = m - A
        for idx, b in enumerate(_BITS):
            s = 1 << b
            bit = (klen >> b) & 1

            def mkc(loff=loff, s=s, idx=idx):
                return pltpu.make_async_copy(
                    sall_ref.at[pl.ds(pl.multiple_of(lbase + loff, 8), s), :],
                    out_ref.at[pl.ds(pl.multiple_of(lbase + loff, 8), s), :],
                    copy_sems.at[idx],
                )

            @pl.when(bit == 1)
            def _(mkc=mkc):
                mkc().start()

            copies.append((bit, mkc))
            loff = loff + bit * s

        pl.semaphore_wait(barrier_sem, 1)

        descs = []
        off = jnp.int32(0)
        for idx, b in enumerate(_BITS):
            s = 1 << b
            bit = (A >> b) & 1

            def mk(off=off, s=s, idx=idx):
                return pltpu.make_async_remote_copy(
                    src_ref=sall_ref.at[pl.ds(pl.multiple_of(m + off, 8), s), :],
                    dst_ref=out_ref.at[
                        pl.ds(pl.multiple_of(dst0 + off, 8), s), :],
                    send_sem=send_sems.at[idx],
                    recv_sem=recv_sems.at[idx],
                    device_id=peer,
                    device_id_type=pl.DeviceIdType.MESH,
                )

            def mkr(off=off, s=s, idx=idx):
                return pltpu.make_async_remote_copy(
                    src_ref=sall_ref.at[pl.ds(pl.multiple_of(m + off, 8), s), :],
                    dst_ref=out_ref.at[
                        pl.ds(pl.multiple_of(rbase + off, 8), s), :],
                    send_sem=send_sems.at[idx],
                    recv_sem=recv_sems.at[idx],
                    device_id=peer,
                    device_id_type=pl.DeviceIdType.MESH,
                )

            @pl.when(bit == 1)
            def _(mk=mk):
                mk().start()

            descs.append((bit, mk, mkr))
            off = off + bit * s

        for bit, mkc in copies:
            @pl.when(bit == 1)
            def _(mkc=mkc):
                mkc().wait()

        for bit, mk, mkr in descs:
            @pl.when(bit == 1)
            def _(mk=mk, mkr=mkr):
                mk().wait_send()
                mkr().wait_recv()

        @pl.when(e > 0)
        def _():
            bb = pl.multiple_of(jnp.where(mz == 0, m - A, A - 8), 8)
            band = out_ref[pl.ds(bb, 8), :]
            keep = sall_ref[pl.ds(bb, 8), :]
            j = lax.broadcasted_iota(jnp.int32, (8, n), 0)
            jj = 7 * mz + (1 - 2 * mz) * j
            out_ref[pl.ds(bb, 8), :] = jnp.where(jj < e, keep, band)

    return pl.pallas_call(
        body,
        out_shape=jax.ShapeDtypeStruct((m, n), x.dtype),
        in_specs=[
            pl.BlockSpec(memory_space=pltpu.SMEM),
            pl.BlockSpec(memory_space=pltpu.VMEM),
        ],
        out_specs=pl.BlockSpec(memory_space=pltpu.VMEM),
        scratch_shapes=[
            pltpu.SemaphoreType.DMA((len(_BITS),)),
            pltpu.SemaphoreType.DMA((len(_BITS),)),
            pltpu.SemaphoreType.DMA((len(_BITS),)),
        ],
        compiler_params=pltpu.CompilerParams(collective_id=0),
    )(a_arr, sall)
